# Initial kernel scaffold; baseline (speedup 1.0000x reference)
import jax
import jax.numpy as jnp
from jax.experimental import pallas as pl


def kernel(x, pointer_addresses, W, b):
    raise NotImplementedError("write your pallas kernel here")


# speedup vs baseline: 1.0747x; 1.0747x over previous
import functools

import jax
import jax.numpy as jnp
from jax.experimental import pallas as pl
from jax.experimental.pallas import tpu as pltpu

E = 8
D = 768
V = 32000
B = 128
VT = 1280
NVT = V // VT


def _router_body(idx_ref, x_ref, w_ref, b_ref, out_ref, xm_ref):
    v = pl.program_id(0)
    e = pl.program_id(1)

    idx = idx_ref[...] % E

    @pl.when(v == 0)
    def _():
        mask = idx == e
        xm = jnp.where(mask, x_ref[...], 0.0)
        xm_ref[e] = xm.astype(jnp.bfloat16)

    xmb = xm_ref[e]
    wb = w_ref[0].astype(jnp.bfloat16)
    acc = jax.lax.dot_general(
        xmb, wb,
        dimension_numbers=(((1,), (1,)), ((), ())),
        preferred_element_type=jnp.float32,
    )
    sel = (idx == e).astype(jnp.float32)
    acc = acc + sel * b_ref[0, 0][None, :]

    @pl.when(e == 0)
    def _():
        out_ref[...] = acc

    @pl.when(e > 0)
    def _():
        out_ref[...] += acc


@jax.jit
def _router(idx_col, x, W, b):
    grid = (NVT, E)
    return pl.pallas_call(
        _router_body,
        grid=grid,
        in_specs=[
            pl.BlockSpec((B, 1), lambda v, e: (0, 0)),
            pl.BlockSpec((B, D), lambda v, e: (0, 0)),
            pl.BlockSpec((1, VT, D), lambda v, e: (e, v, 0)),
            pl.BlockSpec((1, 1, VT), lambda v, e: (e, 0, v)),
        ],
        out_specs=pl.BlockSpec((B, VT), lambda v, e: (0, v)),
        out_shape=jax.ShapeDtypeStruct((B, V), jnp.float32),
        scratch_shapes=[pltpu.VMEM((E, B, D), jnp.bfloat16)],
    )(idx_col, x, W, b.reshape(E, 1, V))


def kernel(x, pointer_addresses, W, b):
    ptr_col = pointer_addresses.astype(jnp.int32).reshape(B, 1)
    return _router(ptr_col, x, W, b)


# speedup vs baseline: 1.4075x; 1.3096x over previous
import functools

import jax
import jax.numpy as jnp
from jax.experimental import pallas as pl
from jax.experimental.pallas import tpu as pltpu

E = 8
D = 768
V = 32000
B = 128
VT = 3200
NVT = V // VT


def _router_body(idx_ref, x_ref, w_ref, b_ref, out_ref, xm_ref, oh_ref):
    v = pl.program_id(0)
    e = pl.program_id(1)

    @pl.when(v == 0)
    def _():
        idx = idx_ref[...] % E
        mask = idx == e
        xm = jnp.where(mask, x_ref[...], 0.0)
        xm_ref[e] = xm.astype(jnp.bfloat16)

        @pl.when(e == 0)
        def _():
            cols = jax.lax.broadcasted_iota(jnp.int32, (B, E), 1)
            oh_ref[...] = (idx == cols).astype(jnp.bfloat16)

    xmb = xm_ref[e]
    wb = w_ref[0].astype(jnp.bfloat16)
    acc = jax.lax.dot_general(
        xmb, wb,
        dimension_numbers=(((1,), (1,)), ((), ())),
        preferred_element_type=jnp.float32,
    )

    @pl.when(e == 0)
    def _():
        bias = jax.lax.dot_general(
            oh_ref[...], b_ref[...].astype(jnp.bfloat16),
            dimension_numbers=(((1,), (0,)), ((), ())),
            preferred_element_type=jnp.float32,
        )
        out_ref[...] = acc + bias

    @pl.when(e > 0)
    def _():
        out_ref[...] += acc


@jax.jit
def _router(idx_col, x, W, b):
    grid = (NVT, E)
    return pl.pallas_call(
        _router_body,
        grid=grid,
        in_specs=[
            pl.BlockSpec((B, 1), lambda v, e: (0, 0)),
            pl.BlockSpec((B, D), lambda v, e: (0, 0)),
            pl.BlockSpec((1, VT, D), lambda v, e: (e, v, 0)),
            pl.BlockSpec((E, VT), lambda v, e: (0, v)),
        ],
        out_specs=pl.BlockSpec((B, VT), lambda v, e: (0, v)),
        out_shape=jax.ShapeDtypeStruct((B, V), jnp.float32),
        scratch_shapes=[
            pltpu.VMEM((E, B, D), jnp.bfloat16),
            pltpu.VMEM((B, E), jnp.bfloat16),
        ],
    )(idx_col, x, W, b)


def kernel(x, pointer_addresses, W, b):
    ptr_col = pointer_addresses.astype(jnp.int32).reshape(B, 1)
    return _router(ptr_col, x, W, b)


# speedup vs baseline: 1.4526x; 1.0321x over previous
import functools

import jax
import jax.numpy as jnp
from jax.experimental import pallas as pl
from jax.experimental.pallas import tpu as pltpu

E = 8
D = 768
V = 32000
B = 128
VT = 6400
NVT = V // VT


def _router_body(idx_ref, x_ref, w_ref, b_ref, out_ref, xm_ref, oh_ref):
    v = pl.program_id(0)
    e = pl.program_id(1)

    @pl.when(v == 0)
    def _():
        idx = idx_ref[...] % E
        mask = idx == e
        xm = jnp.where(mask, x_ref[...], 0.0)
        xm_ref[e] = xm.astype(jnp.bfloat16)

        @pl.when(e == 0)
        def _():
            cols = jax.lax.broadcasted_iota(jnp.int32, (B, E), 1)
            oh_ref[...] = (idx == cols).astype(jnp.bfloat16)

    xmb = xm_ref[e]
    wb = w_ref[0].astype(jnp.bfloat16)
    acc = jax.lax.dot_general(
        xmb, wb,
        dimension_numbers=(((1,), (1,)), ((), ())),
        preferred_element_type=jnp.float32,
    )

    @pl.when(e == 0)
    def _():
        bias = jax.lax.dot_general(
            oh_ref[...], b_ref[...].astype(jnp.bfloat16),
            dimension_numbers=(((1,), (0,)), ((), ())),
            preferred_element_type=jnp.float32,
        )
        out_ref[...] = acc + bias

    @pl.when(e > 0)
    def _():
        out_ref[...] += acc


@jax.jit
def _router(idx_col, x, W, b):
    grid = (NVT, E)
    return pl.pallas_call(
        _router_body,
        grid=grid,
        in_specs=[
            pl.BlockSpec((B, 1), lambda v, e: (0, 0)),
            pl.BlockSpec((B, D), lambda v, e: (0, 0)),
            pl.BlockSpec((1, VT, D), lambda v, e: (e, v, 0)),
            pl.BlockSpec((E, VT), lambda v, e: (0, v)),
        ],
        out_specs=pl.BlockSpec((B, VT), lambda v, e: (0, v)),
        out_shape=jax.ShapeDtypeStruct((B, V), jnp.float32),
        scratch_shapes=[
            pltpu.VMEM((E, B, D), jnp.bfloat16),
            pltpu.VMEM((B, E), jnp.bfloat16),
        ],
    )(idx_col, x, W, b)


def kernel(x, pointer_addresses, W, b):
    ptr_col = pointer_addresses.astype(jnp.int32).reshape(B, 1)
    return _router(ptr_col, x, W, b)
